# baseline (speedup 1.0000x reference)
import functools

import jax
import jax.numpy as jnp
from jax import lax
from jax.experimental import pallas as pl
from jax.experimental.pallas import tpu as pltpu
from jax.experimental.pallas import tpu_sc as plsc

TOP_K = 2
BLK_M = 128
TOK_BLK = 256
LANE = 128


def _sigmoid(v):
    return 1.0 / (1.0 + jnp.exp(-v))


def _gather_rows(table, idx):
    V, D = table.shape
    (B,) = idx.shape
    info = plsc.get_sparse_core_info()
    nw = info.num_cores * info.num_subcores
    assert D % info.num_lanes == 0 and B % (8 * nw) == 0
    b_per_w = B // nw
    mesh = plsc.VectorSubcoreMesh(core_axis_name="c", subcore_axis_name="s")

    @functools.partial(
        pl.kernel,
        mesh=mesh,
        out_type=jax.ShapeDtypeStruct((B, D), jnp.float32),
        scratch_types=[
            pltpu.VMEM((b_per_w,), jnp.int32),
            pltpu.VMEM((b_per_w, D), jnp.float32),
            pltpu.SemaphoreType.DMA,
        ],
    )
    def k(table_hbm, idx_hbm, out_hbm, idx_v, rows_v, sem):
        wid = lax.axis_index("s") * info.num_cores + lax.axis_index("c")
        base = wid * b_per_w
        pltpu.sync_copy(idx_hbm.at[pl.ds(base, b_per_w)], idx_v)
        pltpu.async_copy(table_hbm.at[idx_v], rows_v, sem).wait()
        pltpu.sync_copy(rows_v, out_hbm.at[pl.ds(base, b_per_w)])

    return k(table, idx)


def kernel(x, Wg, W1, W3, W2, Ws1, Ws3, Ws2):
    Bb, Tt, C = x.shape
    S = Bb * Tt
    E = Wg.shape[1]
    H = W1.shape[2]
    N_BUF = S * TOP_K + E * BLK_M
    N_BLK = N_BUF // BLK_M
    f32 = jnp.float32

    x_flat = x.reshape(S, C)

    wg_pad = jnp.pad(Wg, ((0, 0), (0, LANE - E)))

    def _router_body(x_ref, wg_ref, w_ref, i_ref, aux_ref):
        xb = x_ref[...]
        logits = jnp.dot(xb, wg_ref[...], preferred_element_type=f32)
        col = lax.broadcasted_iota(jnp.int32, logits.shape, 1)
        valid = col < E
        ml = jnp.where(valid, logits, -1e30)
        m = jnp.max(ml, axis=1, keepdims=True)
        p = jnp.where(valid, jnp.exp(ml - m), 0.0)
        rw = p / jnp.sum(p, axis=1, keepdims=True)
        w0 = jnp.max(rw, axis=1, keepdims=True)
        is0 = jnp.logical_and(rw == w0, valid)
        e0 = jnp.min(jnp.where(is0, col, E), axis=1, keepdims=True)
        rwm = jnp.where(col == e0, -1.0, rw)
        w1v = jnp.max(rwm, axis=1, keepdims=True)
        is1 = jnp.logical_and(rwm == w1v,
                              jnp.logical_and(valid, col != e0))
        e1 = jnp.min(jnp.where(is1, col, E), axis=1, keepdims=True)
        imp = jnp.sum(rw, axis=0, keepdims=True)
        ld = jnp.sum(jnp.where(col == e0, 1.0, 0.0), axis=0, keepdims=True)
        aux = E * jnp.sum(imp * ld) / (S * S)
        aux_ref[...] = jnp.full((8, LANE), aux, dtype=f32)
        sw = w0 + w1v
        w_ref[...] = jnp.where(col == 0, w0 / sw,
                               jnp.where(col == 1, w1v / sw, 0.0))
        i_ref[...] = jnp.where(col == 0, e0,
                               jnp.where(col == 1, e1, 0)).astype(jnp.int32)

    w_pad, i_pad, aux2 = pl.pallas_call(
        _router_body,
        out_shape=[
            jax.ShapeDtypeStruct((S, LANE), f32),
            jax.ShapeDtypeStruct((S, LANE), jnp.int32),
            jax.ShapeDtypeStruct((8, LANE), f32),
        ],
    )(x_flat, wg_pad)
    aux_loss = aux2[0, 0]

    e0 = i_pad[:, 0]
    e1 = i_pad[:, 1]
    earange = jnp.arange(E, dtype=jnp.int32)
    oh0 = (e0[:, None] == earange[None, :]).astype(jnp.int32)
    oh1 = (e1[:, None] == earange[None, :]).astype(jnp.int32)
    oh = oh0 + oh1
    cum = jnp.cumsum(oh, axis=0)
    excl = cum - oh
    counts = cum[-1]
    blk_per_e = (counts + BLK_M - 1) // BLK_M
    ends = jnp.cumsum(blk_per_e)
    starts_rows = (ends - blk_per_e) * BLK_M
    slot0 = starts_rows[None, :] + excl
    pos0 = jnp.sum(slot0 * oh0, axis=1).astype(jnp.int32)
    pos1 = jnp.sum(slot0 * oh1, axis=1).astype(jnp.int32)
    tid = jnp.arange(S, dtype=jnp.int32)
    base_tid = jnp.arange(N_BUF, dtype=jnp.int32) % S
    sorted_tid = base_tid.at[pos0].set(tid).at[pos1].set(tid)
    blk_arange = jnp.arange(N_BLK, dtype=jnp.int32)
    blk_expert = jnp.minimum(
        jnp.sum((ends[None, :] <= blk_arange[:, None]).astype(jnp.int32),
                axis=1),
        E - 1).astype(jnp.int32)

    GBLK = 512
    tid3 = sorted_tid.reshape(N_BUF // GBLK, 1, GBLK)

    def _gather_body(tid_ref, x_ref, o_ref):
        tids = tid_ref[0, 0].reshape(GBLK, 1)
        col = lax.broadcasted_iota(jnp.int32, (GBLK, S), 1)
        oh_g = (col == tids).astype(f32)
        o_ref[...] = jnp.dot(oh_g, x_ref[...], preferred_element_type=f32)

    x_sorted = pl.pallas_call(
        _gather_body,
        grid=(N_BUF // GBLK,),
        in_specs=[
            pl.BlockSpec((1, 1, GBLK), lambda m: (m, 0, 0)),
            pl.BlockSpec((S, C), lambda m: (0, 0)),
        ],
        out_specs=pl.BlockSpec((GBLK, C), lambda m: (m, 0)),
        out_shape=jax.ShapeDtypeStruct((N_BUF, C), f32),
    )(tid3, x_flat)

    def _expert_body(be_ref, xs_ref, w1_ref, w3_ref, w2_ref, o_ref):
        xb = xs_ref[...]
        h1 = jnp.dot(xb, w1_ref[0], preferred_element_type=f32)
        h3 = jnp.dot(xb, w3_ref[0], preferred_element_type=f32)
        glu = h1 * _sigmoid(h1) * h3
        o_ref[...] = jnp.dot(glu, w2_ref[0], preferred_element_type=f32)

    grid_spec = pltpu.PrefetchScalarGridSpec(
        num_scalar_prefetch=1,
        grid=(N_BLK,),
        in_specs=[
            pl.BlockSpec((BLK_M, C), lambda m, be: (m, 0)),
            pl.BlockSpec((1, C, H), lambda m, be: (be[m], 0, 0)),
            pl.BlockSpec((1, C, H), lambda m, be: (be[m], 0, 0)),
            pl.BlockSpec((1, H, C), lambda m, be: (be[m], 0, 0)),
        ],
        out_specs=pl.BlockSpec((BLK_M, C), lambda m, be: (m, 0)),
    )
    y_sorted = pl.pallas_call(
        _expert_body,
        grid_spec=grid_spec,
        out_shape=jax.ShapeDtypeStruct((N_BUF, C), f32),
    )(blk_expert, x_sorted, W1, W3, W2)

    pos01 = jnp.concatenate([pos0, pos1])
    g = _gather_rows(y_sorted, pos01)

    def _shared_body(x_ref, ws1_ref, ws3_ref, ws2_ref, o_ref):
        xb = x_ref[...]
        h1 = jnp.dot(xb, ws1_ref[...], preferred_element_type=f32)
        h3 = jnp.dot(xb, ws3_ref[...], preferred_element_type=f32)
        glu = h1 * _sigmoid(h1) * h3
        o_ref[...] = jnp.dot(glu, ws2_ref[...], preferred_element_type=f32)

    shared_out = pl.pallas_call(
        _shared_body,
        grid=(S // TOK_BLK,),
        in_specs=[
            pl.BlockSpec((TOK_BLK, C), lambda m: (m, 0)),
            pl.BlockSpec((C, H), lambda m: (0, 0)),
            pl.BlockSpec((C, H), lambda m: (0, 0)),
            pl.BlockSpec((H, C), lambda m: (0, 0)),
        ],
        out_specs=pl.BlockSpec((TOK_BLK, C), lambda m: (m, 0)),
        out_shape=jax.ShapeDtypeStruct((S, C), f32),
    )(x_flat, Ws1, Ws3, Ws2)

    def _combine_body(sh_ref, g0_ref, g1_ref, wf_ref, o_ref):
        wf = wf_ref[...]
        o_ref[...] = (sh_ref[...] + wf[:, 0:1] * g0_ref[...]
                      + wf[:, 1:2] * g1_ref[...])

    out = pl.pallas_call(
        _combine_body,
        grid=(S // TOK_BLK,),
        in_specs=[
            pl.BlockSpec((TOK_BLK, C), lambda m: (m, 0)),
            pl.BlockSpec((TOK_BLK, C), lambda m: (m, 0)),
            pl.BlockSpec((TOK_BLK, C), lambda m: (m + S // TOK_BLK, 0)),
            pl.BlockSpec((TOK_BLK, LANE), lambda m: (m, 0)),
        ],
        out_specs=pl.BlockSpec((TOK_BLK, C), lambda m: (m, 0)),
        out_shape=jax.ShapeDtypeStruct((S, C), f32),
    )(shared_out, g, g, w_pad)

    return out.reshape(Bb, Tt, C), aux_loss


# speedup vs baseline: 1.4765x; 1.4765x over previous
import functools

import jax
import jax.numpy as jnp
from jax import lax
from jax.experimental import pallas as pl
from jax.experimental.pallas import tpu as pltpu
from jax.experimental.pallas import tpu_sc as plsc

TOP_K = 2
BLK_M = 128
TOK_BLK = 256
LANE = 128


def _sigmoid(v):
    return 1.0 / (1.0 + jnp.exp(-v))


def _gather_rows(table, idx):
    V, D = table.shape
    (B,) = idx.shape
    info = plsc.get_sparse_core_info()
    nw = info.num_cores * info.num_subcores
    assert D % info.num_lanes == 0 and B % (8 * nw) == 0
    b_per_w = B // nw
    mesh = plsc.VectorSubcoreMesh(core_axis_name="c", subcore_axis_name="s")

    @functools.partial(
        pl.kernel,
        mesh=mesh,
        out_type=jax.ShapeDtypeStruct((B, D), jnp.float32),
        scratch_types=[
            pltpu.VMEM((b_per_w,), jnp.int32),
            pltpu.VMEM((b_per_w, D), jnp.float32),
            pltpu.SemaphoreType.DMA,
        ],
    )
    def k(table_hbm, idx_hbm, out_hbm, idx_v, rows_v, sem):
        wid = lax.axis_index("s") * info.num_cores + lax.axis_index("c")
        base = wid * b_per_w
        pltpu.sync_copy(idx_hbm.at[pl.ds(base, b_per_w)], idx_v)
        pltpu.async_copy(table_hbm.at[idx_v], rows_v, sem).wait()
        pltpu.sync_copy(rows_v, out_hbm.at[pl.ds(base, b_per_w)])

    return k(table, idx)


def kernel(x, Wg, W1, W3, W2, Ws1, Ws3, Ws2):
    Bb, Tt, C = x.shape
    S = Bb * Tt
    E = Wg.shape[1]
    H = W1.shape[2]
    N_BUF = S * TOP_K + E * BLK_M
    N_BLK = N_BUF // BLK_M
    f32 = jnp.float32

    x_flat = x.reshape(S, C)

    wg_pad = jnp.pad(Wg, ((0, 0), (0, LANE - E)))

    def _router_body(x_ref, wg_ref, w_ref, i_ref, aux_ref):
        xb = x_ref[...]
        logits = jnp.dot(xb, wg_ref[...], preferred_element_type=f32)
        col = lax.broadcasted_iota(jnp.int32, logits.shape, 1)
        valid = col < E
        ml = jnp.where(valid, logits, -1e30)
        m = jnp.max(ml, axis=1, keepdims=True)
        p = jnp.where(valid, jnp.exp(ml - m), 0.0)
        rw = p / jnp.sum(p, axis=1, keepdims=True)
        w0 = jnp.max(rw, axis=1, keepdims=True)
        is0 = jnp.logical_and(rw == w0, valid)
        e0 = jnp.min(jnp.where(is0, col, E), axis=1, keepdims=True)
        rwm = jnp.where(col == e0, -1.0, rw)
        w1v = jnp.max(rwm, axis=1, keepdims=True)
        is1 = jnp.logical_and(rwm == w1v,
                              jnp.logical_and(valid, col != e0))
        e1 = jnp.min(jnp.where(is1, col, E), axis=1, keepdims=True)
        imp = jnp.sum(rw, axis=0, keepdims=True)
        ld = jnp.sum(jnp.where(col == e0, 1.0, 0.0), axis=0, keepdims=True)
        aux = E * jnp.sum(imp * ld) / (S * S)
        aux_ref[...] = jnp.full((8, LANE), aux, dtype=f32)
        sw = w0 + w1v
        w_ref[...] = jnp.where(col == 0, w0 / sw,
                               jnp.where(col == 1, w1v / sw, 0.0))
        i_ref[...] = jnp.where(col == 0, e0,
                               jnp.where(col == 1, e1, 0)).astype(jnp.int32)

    w_pad, i_pad, aux2 = pl.pallas_call(
        _router_body,
        out_shape=[
            jax.ShapeDtypeStruct((S, LANE), f32),
            jax.ShapeDtypeStruct((S, LANE), jnp.int32),
            jax.ShapeDtypeStruct((8, LANE), f32),
        ],
    )(x_flat, wg_pad)
    aux_loss = aux2[0, 0]

    e0 = i_pad[:, 0]
    e1 = i_pad[:, 1]
    earange = jnp.arange(E, dtype=jnp.int32)
    oh0 = (e0[:, None] == earange[None, :]).astype(jnp.int32)
    oh1 = (e1[:, None] == earange[None, :]).astype(jnp.int32)
    oh = oh0 + oh1
    cum = jnp.cumsum(oh, axis=0)
    excl = cum - oh
    counts = cum[-1]
    blk_per_e = (counts + BLK_M - 1) // BLK_M
    ends = jnp.cumsum(blk_per_e)
    starts_rows = (ends - blk_per_e) * BLK_M
    slot0 = starts_rows[None, :] + excl
    pos0 = jnp.sum(slot0 * oh0, axis=1).astype(jnp.int32)
    pos1 = jnp.sum(slot0 * oh1, axis=1).astype(jnp.int32)
    tid = jnp.arange(S, dtype=jnp.int32)
    base_tid = jnp.arange(N_BUF, dtype=jnp.int32) % S
    sorted_tid = base_tid.at[pos0].set(tid).at[pos1].set(tid)
    blk_arange = jnp.arange(N_BLK, dtype=jnp.int32)
    blk_expert = jnp.minimum(
        jnp.sum((ends[None, :] <= blk_arange[:, None]).astype(jnp.int32),
                axis=1),
        E - 1).astype(jnp.int32)
    chg = jnp.concatenate([
        jnp.ones((1,), jnp.int32),
        (blk_expert[1:] != blk_expert[:-1]).astype(jnp.int32)])
    ordn = jnp.cumsum(chg) - 1
    cand = jnp.where(chg == 1, blk_arange, 2 * N_BLK)
    suffix_min = jnp.flip(jax.lax.cummin(jnp.flip(cand)))
    nxt_pos = jnp.concatenate(
        [suffix_min[1:], jnp.full((1,), 2 * N_BLK, jnp.int32)])
    has_nxt = (nxt_pos < N_BLK).astype(jnp.int32)
    nxt_e = jnp.sum(
        (nxt_pos[:, None] == blk_arange[None, :]).astype(jnp.int32)
        * blk_expert[None, :], axis=1)
    pf = jnp.stack([chg, has_nxt, nxt_e, ordn, blk_expert]).astype(jnp.int32)

    GBLK = 512
    tid3 = sorted_tid.reshape(N_BUF // GBLK, 1, GBLK)

    def _gather_body(tid_ref, x_ref, o_ref):
        tids = tid_ref[0, 0].reshape(GBLK, 1)
        col = lax.broadcasted_iota(jnp.int32, (GBLK, S), 1)
        oh_g = (col == tids).astype(f32)
        o_ref[...] = jnp.dot(oh_g, x_ref[...], preferred_element_type=f32)

    x_sorted = pl.pallas_call(
        _gather_body,
        grid=(N_BUF // GBLK,),
        in_specs=[
            pl.BlockSpec((1, 1, GBLK), lambda m: (m, 0, 0)),
            pl.BlockSpec((S, C), lambda m: (0, 0)),
        ],
        out_specs=pl.BlockSpec((GBLK, C), lambda m: (m, 0)),
        out_shape=jax.ShapeDtypeStruct((N_BUF, C), f32),
    )(tid3, x_flat)

    w1t = jnp.transpose(W1, (2, 0, 1))
    w3t = jnp.transpose(W3, (2, 0, 1))
    w2t = jnp.transpose(W2, (1, 0, 2))

    def _expert_body(pf_ref, xs_ref, w1_hbm, w3_hbm, w2_hbm, o_ref,
                     w1b, w3b, w2b, sems):
        m = pl.program_id(0)
        slot = pf_ref[3, m] % 2

        def _issue(e, s):
            pltpu.make_async_copy(w1_hbm.at[:, e, :], w1b.at[s],
                                  sems.at[s]).start()
            pltpu.make_async_copy(w3_hbm.at[:, e, :], w3b.at[s],
                                  sems.at[s]).start()
            pltpu.make_async_copy(w2_hbm.at[:, e, :], w2b.at[s],
                                  sems.at[s]).start()

        def _wait(s):
            pltpu.make_async_copy(w1_hbm.at[:, 0, :], w1b.at[s],
                                  sems.at[s]).wait()
            pltpu.make_async_copy(w3_hbm.at[:, 0, :], w3b.at[s],
                                  sems.at[s]).wait()
            pltpu.make_async_copy(w2_hbm.at[:, 0, :], w2b.at[s],
                                  sems.at[s]).wait()

        @pl.when(m == 0)
        def _():
            _issue(pf_ref[4, 0], 0)

        @pl.when(pf_ref[0, m] == 1)
        def _():
            _wait(slot)

            @pl.when(pf_ref[1, m] == 1)
            def _():
                _issue(pf_ref[2, m], 1 - slot)

        xb = xs_ref[...]
        dn_t = (((1,), (1,)), ((), ()))
        dn_n = (((1,), (0,)), ((), ()))
        h1 = lax.dot_general(xb, w1b[slot], dn_t,
                             preferred_element_type=f32)
        h3 = lax.dot_general(xb, w3b[slot], dn_t,
                             preferred_element_type=f32)
        glu = h1 * _sigmoid(h1) * h3
        o_ref[...] = lax.dot_general(glu, w2b[slot], dn_n,
                                     preferred_element_type=f32)

    grid_spec = pltpu.PrefetchScalarGridSpec(
        num_scalar_prefetch=1,
        grid=(N_BLK,),
        in_specs=[
            pl.BlockSpec((BLK_M, C), lambda m, pfr: (m, 0)),
            pl.BlockSpec(memory_space=pl.ANY),
            pl.BlockSpec(memory_space=pl.ANY),
            pl.BlockSpec(memory_space=pl.ANY),
        ],
        out_specs=pl.BlockSpec((BLK_M, C), lambda m, pfr: (m, 0)),
        scratch_shapes=[
            pltpu.VMEM((2, H, C), f32),
            pltpu.VMEM((2, H, C), f32),
            pltpu.VMEM((2, H, C), f32),
            pltpu.SemaphoreType.DMA((2,)),
        ],
    )
    y_sorted = pl.pallas_call(
        _expert_body,
        grid_spec=grid_spec,
        out_shape=jax.ShapeDtypeStruct((N_BUF, C), f32),
    )(pf, x_sorted, w1t, w3t, w2t)

    pos01 = jnp.concatenate([pos0, pos1])
    g = _gather_rows(y_sorted, pos01)

    ws1t = Ws1.T
    ws3t = Ws3.T

    def _shared_body(x_ref, ws1_ref, ws3_ref, ws2_ref, o_ref):
        xb = x_ref[...]
        dn_t = (((1,), (1,)), ((), ()))
        h1 = lax.dot_general(xb, ws1_ref[...], dn_t,
                             preferred_element_type=f32)
        h3 = lax.dot_general(xb, ws3_ref[...], dn_t,
                             preferred_element_type=f32)
        glu = h1 * _sigmoid(h1) * h3
        o_ref[...] = jnp.dot(glu, ws2_ref[...], preferred_element_type=f32)

    shared_out = pl.pallas_call(
        _shared_body,
        grid=(S // TOK_BLK,),
        in_specs=[
            pl.BlockSpec((TOK_BLK, C), lambda m: (m, 0)),
            pl.BlockSpec((H, C), lambda m: (0, 0)),
            pl.BlockSpec((H, C), lambda m: (0, 0)),
            pl.BlockSpec((H, C), lambda m: (0, 0)),
        ],
        out_specs=pl.BlockSpec((TOK_BLK, C), lambda m: (m, 0)),
        out_shape=jax.ShapeDtypeStruct((S, C), f32),
    )(x_flat, ws1t, ws3t, Ws2)

    def _combine_body(sh_ref, g0_ref, g1_ref, wf_ref, o_ref):
        wf = wf_ref[...]
        o_ref[...] = (sh_ref[...] + wf[:, 0:1] * g0_ref[...]
                      + wf[:, 1:2] * g1_ref[...])

    out = pl.pallas_call(
        _combine_body,
        grid=(S // TOK_BLK,),
        in_specs=[
            pl.BlockSpec((TOK_BLK, C), lambda m: (m, 0)),
            pl.BlockSpec((TOK_BLK, C), lambda m: (m, 0)),
            pl.BlockSpec((TOK_BLK, C), lambda m: (m + S // TOK_BLK, 0)),
            pl.BlockSpec((TOK_BLK, LANE), lambda m: (m, 0)),
        ],
        out_specs=pl.BlockSpec((TOK_BLK, C), lambda m: (m, 0)),
        out_shape=jax.ShapeDtypeStruct((S, C), f32),
    )(shared_out, g, g, w_pad)

    return out.reshape(Bb, Tt, C), aux_loss


# speedup vs baseline: 1.5930x; 1.0789x over previous
import functools

import jax
import jax.numpy as jnp
from jax import lax
from jax.experimental import pallas as pl
from jax.experimental.pallas import tpu as pltpu
from jax.experimental.pallas import tpu_sc as plsc

TOP_K = 2
BLK_M = 128
TOK_BLK = 256
LANE = 128


def _sigmoid(v):
    return 1.0 / (1.0 + jnp.exp(-v))


def _gather_rows(table, idx):
    V, D = table.shape
    (B,) = idx.shape
    info = plsc.get_sparse_core_info()
    nw = info.num_cores * info.num_subcores
    assert D % info.num_lanes == 0 and B % (8 * nw) == 0
    b_per_w = B // nw
    mesh = plsc.VectorSubcoreMesh(core_axis_name="c", subcore_axis_name="s")

    @functools.partial(
        pl.kernel,
        mesh=mesh,
        out_type=jax.ShapeDtypeStruct((B, D), jnp.float32),
        scratch_types=[
            pltpu.VMEM((b_per_w,), jnp.int32),
            pltpu.VMEM((b_per_w, D), jnp.float32),
            pltpu.SemaphoreType.DMA,
        ],
    )
    def k(table_hbm, idx_hbm, out_hbm, idx_v, rows_v, sem):
        wid = lax.axis_index("s") * info.num_cores + lax.axis_index("c")
        base = wid * b_per_w
        pltpu.sync_copy(idx_hbm.at[pl.ds(base, b_per_w)], idx_v)
        pltpu.async_copy(table_hbm.at[idx_v], rows_v, sem).wait()
        pltpu.sync_copy(rows_v, out_hbm.at[pl.ds(base, b_per_w)])

    return k(table, idx)


def kernel(x, Wg, W1, W3, W2, Ws1, Ws3, Ws2):
    Bb, Tt, C = x.shape
    S = Bb * Tt
    E = Wg.shape[1]
    H = W1.shape[2]
    N_BUF = S * TOP_K + E * BLK_M
    N_BLK = N_BUF // BLK_M
    f32 = jnp.float32

    x_flat = x.reshape(S, C)

    wg_pad = jnp.pad(Wg, ((0, 0), (0, LANE - E)))

    def _router_body(x_ref, wg_ref, w_ref, i_ref, aux_ref):
        xb = x_ref[...]
        logits = jnp.dot(xb, wg_ref[...], preferred_element_type=f32)
        col = lax.broadcasted_iota(jnp.int32, logits.shape, 1)
        valid = col < E
        ml = jnp.where(valid, logits, -1e30)
        m = jnp.max(ml, axis=1, keepdims=True)
        p = jnp.where(valid, jnp.exp(ml - m), 0.0)
        rw = p / jnp.sum(p, axis=1, keepdims=True)
        w0 = jnp.max(rw, axis=1, keepdims=True)
        is0 = jnp.logical_and(rw == w0, valid)
        e0 = jnp.min(jnp.where(is0, col, E), axis=1, keepdims=True)
        rwm = jnp.where(col == e0, -1.0, rw)
        w1v = jnp.max(rwm, axis=1, keepdims=True)
        is1 = jnp.logical_and(rwm == w1v,
                              jnp.logical_and(valid, col != e0))
        e1 = jnp.min(jnp.where(is1, col, E), axis=1, keepdims=True)
        imp = jnp.sum(rw, axis=0, keepdims=True)
        ld = jnp.sum(jnp.where(col == e0, 1.0, 0.0), axis=0, keepdims=True)
        aux = E * jnp.sum(imp * ld) / (S * S)
        aux_ref[...] = jnp.full((8, LANE), aux, dtype=f32)
        sw = w0 + w1v
        w_ref[...] = jnp.where(col == 0, w0 / sw,
                               jnp.where(col == 1, w1v / sw, 0.0))
        i_ref[...] = jnp.where(col == 0, e0,
                               jnp.where(col == 1, e1, 0)).astype(jnp.int32)

    w_pad, i_pad, aux2 = pl.pallas_call(
        _router_body,
        out_shape=[
            jax.ShapeDtypeStruct((S, LANE), f32),
            jax.ShapeDtypeStruct((S, LANE), jnp.int32),
            jax.ShapeDtypeStruct((8, LANE), f32),
        ],
    )(x_flat, wg_pad)
    aux_loss = aux2[0, 0]

    e0 = i_pad[:, 0]
    e1 = i_pad[:, 1]
    earange = jnp.arange(E, dtype=jnp.int32)
    oh0 = (e0[:, None] == earange[None, :]).astype(jnp.int32)
    oh1 = (e1[:, None] == earange[None, :]).astype(jnp.int32)
    oh = oh0 + oh1
    cum = jnp.cumsum(oh, axis=0)
    excl = cum - oh
    counts = cum[-1]
    blk_per_e = (counts + BLK_M - 1) // BLK_M
    ends = jnp.cumsum(blk_per_e)
    starts_rows = (ends - blk_per_e) * BLK_M
    slot0 = starts_rows[None, :] + excl
    pos0 = jnp.sum(slot0 * oh0, axis=1).astype(jnp.int32)
    pos1 = jnp.sum(slot0 * oh1, axis=1).astype(jnp.int32)
    blk_arange = jnp.arange(N_BLK, dtype=jnp.int32)
    blk_expert = jnp.minimum(
        jnp.sum((ends[None, :] <= blk_arange[:, None]).astype(jnp.int32),
                axis=1),
        E - 1).astype(jnp.int32)
    chg = jnp.concatenate([
        jnp.ones((1,), jnp.int32),
        (blk_expert[1:] != blk_expert[:-1]).astype(jnp.int32)])
    ordn = jnp.cumsum(chg) - 1
    cand = jnp.where(chg == 1, blk_arange, 2 * N_BLK)
    suffix_min = jnp.flip(jax.lax.cummin(jnp.flip(cand)))
    nxt_pos = jnp.concatenate(
        [suffix_min[1:], jnp.full((1,), 2 * N_BLK, jnp.int32)])
    has_nxt = (nxt_pos < N_BLK).astype(jnp.int32)
    nxt_e = jnp.sum(
        (nxt_pos[:, None] == blk_arange[None, :]).astype(jnp.int32)
        * blk_expert[None, :], axis=1)
    pf = jnp.stack([chg, has_nxt, nxt_e, ordn, blk_expert]).astype(jnp.int32)

    GBLK = 512
    pos0r = pos0.reshape(1, S)
    pos1r = pos1.reshape(1, S)

    def _gather_body(p0_ref, p1_ref, x_ref, o_ref):
        m = pl.program_id(0)
        slot = lax.broadcasted_iota(jnp.int32, (GBLK, S), 0) + m * GBLK
        oh_g = jnp.logical_or(slot == p0_ref[...],
                              slot == p1_ref[...]).astype(f32)
        o_ref[...] = jnp.dot(oh_g, x_ref[...], preferred_element_type=f32)

    x_sorted = pl.pallas_call(
        _gather_body,
        grid=(N_BUF // GBLK,),
        in_specs=[
            pl.BlockSpec((1, S), lambda m: (0, 0)),
            pl.BlockSpec((1, S), lambda m: (0, 0)),
            pl.BlockSpec((S, C), lambda m: (0, 0)),
        ],
        out_specs=pl.BlockSpec((GBLK, C), lambda m: (m, 0)),
        out_shape=jax.ShapeDtypeStruct((N_BUF, C), f32),
    )(pos0r, pos1r, x_flat)

    w1t = jnp.transpose(W1, (2, 0, 1))
    w3t = jnp.transpose(W3, (2, 0, 1))
    w2t = jnp.transpose(W2, (1, 0, 2))

    def _expert_body(pf_ref, xs_ref, w1_hbm, w3_hbm, w2_hbm, o_ref,
                     w1b, w3b, w2b, sems):
        m = pl.program_id(0)
        slot = pf_ref[3, m] % 2

        def _issue(e, s):
            pltpu.make_async_copy(w1_hbm.at[:, e, :], w1b.at[s],
                                  sems.at[s]).start()
            pltpu.make_async_copy(w3_hbm.at[:, e, :], w3b.at[s],
                                  sems.at[s]).start()
            pltpu.make_async_copy(w2_hbm.at[:, e, :], w2b.at[s],
                                  sems.at[s]).start()

        def _wait(s):
            pltpu.make_async_copy(w1_hbm.at[:, 0, :], w1b.at[s],
                                  sems.at[s]).wait()
            pltpu.make_async_copy(w3_hbm.at[:, 0, :], w3b.at[s],
                                  sems.at[s]).wait()
            pltpu.make_async_copy(w2_hbm.at[:, 0, :], w2b.at[s],
                                  sems.at[s]).wait()

        @pl.when(m == 0)
        def _():
            _issue(pf_ref[4, 0], 0)

        @pl.when(pf_ref[0, m] == 1)
        def _():
            _wait(slot)

            @pl.when(pf_ref[1, m] == 1)
            def _():
                _issue(pf_ref[2, m], 1 - slot)

        xb = xs_ref[...]
        dn_t = (((1,), (1,)), ((), ()))
        dn_n = (((1,), (0,)), ((), ()))
        h1 = lax.dot_general(xb, w1b[slot], dn_t,
                             preferred_element_type=f32)
        h3 = lax.dot_general(xb, w3b[slot], dn_t,
                             preferred_element_type=f32)
        glu = h1 * _sigmoid(h1) * h3
        o_ref[...] = lax.dot_general(glu, w2b[slot], dn_n,
                                     preferred_element_type=f32)

    grid_spec = pltpu.PrefetchScalarGridSpec(
        num_scalar_prefetch=1,
        grid=(N_BLK,),
        in_specs=[
            pl.BlockSpec((BLK_M, C), lambda m, pfr: (m, 0)),
            pl.BlockSpec(memory_space=pl.ANY),
            pl.BlockSpec(memory_space=pl.ANY),
            pl.BlockSpec(memory_space=pl.ANY),
        ],
        out_specs=pl.BlockSpec((BLK_M, C), lambda m, pfr: (m, 0)),
        scratch_shapes=[
            pltpu.VMEM((2, H, C), f32),
            pltpu.VMEM((2, H, C), f32),
            pltpu.VMEM((2, H, C), f32),
            pltpu.SemaphoreType.DMA((2,)),
        ],
    )
    y_sorted = pl.pallas_call(
        _expert_body,
        grid_spec=grid_spec,
        out_shape=jax.ShapeDtypeStruct((N_BUF, C), f32),
    )(pf, x_sorted, w1t, w3t, w2t)

    pos01 = jnp.concatenate([pos0, pos1])
    g = _gather_rows(y_sorted, pos01)

    ws1t = Ws1.T
    ws3t = Ws3.T

    def _shared_body(x_ref, ws1_ref, ws3_ref, ws2_ref, o_ref):
        xb = x_ref[...]
        dn_t = (((1,), (1,)), ((), ()))
        h1 = lax.dot_general(xb, ws1_ref[...], dn_t,
                             preferred_element_type=f32)
        h3 = lax.dot_general(xb, ws3_ref[...], dn_t,
                             preferred_element_type=f32)
        glu = h1 * _sigmoid(h1) * h3
        o_ref[...] = jnp.dot(glu, ws2_ref[...], preferred_element_type=f32)

    shared_out = pl.pallas_call(
        _shared_body,
        grid=(S // TOK_BLK,),
        in_specs=[
            pl.BlockSpec((TOK_BLK, C), lambda m: (m, 0)),
            pl.BlockSpec((H, C), lambda m: (0, 0)),
            pl.BlockSpec((H, C), lambda m: (0, 0)),
            pl.BlockSpec((H, C), lambda m: (0, 0)),
        ],
        out_specs=pl.BlockSpec((TOK_BLK, C), lambda m: (m, 0)),
        out_shape=jax.ShapeDtypeStruct((S, C), f32),
    )(x_flat, ws1t, ws3t, Ws2)

    def _combine_body(sh_ref, g0_ref, g1_ref, wf_ref, o_ref):
        wf = wf_ref[...]
        o_ref[...] = (sh_ref[...] + wf[:, 0:1] * g0_ref[...]
                      + wf[:, 1:2] * g1_ref[...])

    out = pl.pallas_call(
        _combine_body,
        grid=(S // TOK_BLK,),
        in_specs=[
            pl.BlockSpec((TOK_BLK, C), lambda m: (m, 0)),
            pl.BlockSpec((TOK_BLK, C), lambda m: (m, 0)),
            pl.BlockSpec((TOK_BLK, C), lambda m: (m + S // TOK_BLK, 0)),
            pl.BlockSpec((TOK_BLK, LANE), lambda m: (m, 0)),
        ],
        out_specs=pl.BlockSpec((TOK_BLK, C), lambda m: (m, 0)),
        out_shape=jax.ShapeDtypeStruct((S, C), f32),
    )(shared_out, g, g, w_pad)

    return out.reshape(Bb, Tt, C), aux_loss


# speedup vs baseline: 1.7349x; 1.0891x over previous
import functools

import jax
import jax.numpy as jnp
from jax import lax
from jax.experimental import pallas as pl
from jax.experimental.pallas import tpu as pltpu
from jax.experimental.pallas import tpu_sc as plsc

TOP_K = 2
BLK_M = 128
TOK_BLK = 256
LANE = 128


def _sigmoid(v):
    return 1.0 / (1.0 + jnp.exp(-v))


def _gather_rows(table, idx):
    V, D = table.shape
    (B,) = idx.shape
    info = plsc.get_sparse_core_info()
    nw = info.num_cores * info.num_subcores
    assert D % info.num_lanes == 0 and B % (8 * nw) == 0
    b_per_w = B // nw
    mesh = plsc.VectorSubcoreMesh(core_axis_name="c", subcore_axis_name="s")

    @functools.partial(
        pl.kernel,
        mesh=mesh,
        out_type=jax.ShapeDtypeStruct((B, D), jnp.float32),
        scratch_types=[
            pltpu.VMEM((b_per_w,), jnp.int32),
            pltpu.VMEM((b_per_w, D), jnp.float32),
            pltpu.SemaphoreType.DMA,
        ],
    )
    def k(table_hbm, idx_hbm, out_hbm, idx_v, rows_v, sem):
        wid = lax.axis_index("s") * info.num_cores + lax.axis_index("c")
        base = wid * b_per_w
        pltpu.sync_copy(idx_hbm.at[pl.ds(base, b_per_w)], idx_v)
        pltpu.async_copy(table_hbm.at[idx_v], rows_v, sem).wait()
        pltpu.sync_copy(rows_v, out_hbm.at[pl.ds(base, b_per_w)])

    return k(table, idx)


def kernel(x, Wg, W1, W3, W2, Ws1, Ws3, Ws2):
    Bb, Tt, C = x.shape
    S = Bb * Tt
    E = Wg.shape[1]
    H = W1.shape[2]
    N_BUF = S * TOP_K + E * BLK_M
    N_BLK = N_BUF // BLK_M
    f32 = jnp.float32

    x_flat = x.reshape(S, C)

    wg_pad = jnp.pad(Wg, ((0, 0), (0, LANE - E)))

    def _router_body(x_ref, wg_ref, w_ref, i_ref, aux_ref):
        xb = x_ref[...]
        logits = jnp.dot(xb, wg_ref[...], preferred_element_type=f32)
        col = lax.broadcasted_iota(jnp.int32, logits.shape, 1)
        valid = col < E
        ml = jnp.where(valid, logits, -1e30)
        m = jnp.max(ml, axis=1, keepdims=True)
        p = jnp.where(valid, jnp.exp(ml - m), 0.0)
        rw = p / jnp.sum(p, axis=1, keepdims=True)
        w0 = jnp.max(rw, axis=1, keepdims=True)
        is0 = jnp.logical_and(rw == w0, valid)
        e0 = jnp.min(jnp.where(is0, col, E), axis=1, keepdims=True)
        rwm = jnp.where(col == e0, -1.0, rw)
        w1v = jnp.max(rwm, axis=1, keepdims=True)
        is1 = jnp.logical_and(rwm == w1v,
                              jnp.logical_and(valid, col != e0))
        e1 = jnp.min(jnp.where(is1, col, E), axis=1, keepdims=True)
        imp = jnp.sum(rw, axis=0, keepdims=True)
        ld = jnp.sum(jnp.where(col == e0, 1.0, 0.0), axis=0, keepdims=True)
        aux = E * jnp.sum(imp * ld) / (S * S)
        aux_ref[...] = jnp.full((8, LANE), aux, dtype=f32)
        sw = w0 + w1v
        w_ref[...] = jnp.where(col == 0, w0 / sw,
                               jnp.where(col == 1, w1v / sw, 0.0))
        i_ref[...] = jnp.where(col == 0, e0,
                               jnp.where(col == 1, e1, 0)).astype(jnp.int32)

    w_pad, i_pad, aux2 = pl.pallas_call(
        _router_body,
        out_shape=[
            jax.ShapeDtypeStruct((S, LANE), f32),
            jax.ShapeDtypeStruct((S, LANE), jnp.int32),
            jax.ShapeDtypeStruct((8, LANE), f32),
        ],
    )(x_flat, wg_pad)
    aux_loss = aux2[0, 0]

    e0 = i_pad[:, 0]
    e1 = i_pad[:, 1]
    earange = jnp.arange(E, dtype=jnp.int32)
    oh0 = (e0[:, None] == earange[None, :]).astype(jnp.int32)
    oh1 = (e1[:, None] == earange[None, :]).astype(jnp.int32)
    oh = oh0 + oh1
    cum = jnp.cumsum(oh, axis=0)
    excl = cum - oh
    counts = cum[-1]
    blk_per_e = (counts + BLK_M - 1) // BLK_M
    ends = jnp.cumsum(blk_per_e)
    starts_rows = (ends - blk_per_e) * BLK_M
    slot0 = starts_rows[None, :] + excl
    pos0 = jnp.sum(slot0 * oh0, axis=1).astype(jnp.int32)
    pos1 = jnp.sum(slot0 * oh1, axis=1).astype(jnp.int32)
    blk_arange = jnp.arange(N_BLK, dtype=jnp.int32)
    blk_expert = jnp.minimum(
        jnp.sum((ends[None, :] <= blk_arange[:, None]).astype(jnp.int32),
                axis=1),
        E - 1).astype(jnp.int32)
    chg = jnp.concatenate([
        jnp.ones((1,), jnp.int32),
        (blk_expert[1:] != blk_expert[:-1]).astype(jnp.int32)])
    ordn = jnp.cumsum(chg) - 1
    cand = jnp.where(chg == 1, blk_arange, 2 * N_BLK)
    suffix_min = jnp.flip(jax.lax.cummin(jnp.flip(cand)))
    nxt_pos = jnp.concatenate(
        [suffix_min[1:], jnp.full((1,), 2 * N_BLK, jnp.int32)])
    has_nxt = (nxt_pos < N_BLK).astype(jnp.int32)
    nxt_e = jnp.sum(
        (nxt_pos[:, None] == blk_arange[None, :]).astype(jnp.int32)
        * blk_expert[None, :], axis=1)
    pf = jnp.stack([chg, has_nxt, nxt_e, ordn, blk_expert]).astype(jnp.int32)

    GBLK = 512
    pos0r = pos0.reshape(1, S)
    pos1r = pos1.reshape(1, S)

    def _gather_body(p0_ref, p1_ref, x_ref, o_ref):
        m = pl.program_id(0)
        slot = lax.broadcasted_iota(jnp.int32, (GBLK, S), 0) + m * GBLK
        oh_g = jnp.logical_or(slot == p0_ref[...],
                              slot == p1_ref[...]).astype(f32)
        o_ref[...] = jnp.dot(oh_g, x_ref[...], preferred_element_type=f32)

    x_sorted = pl.pallas_call(
        _gather_body,
        grid=(N_BUF // GBLK,),
        in_specs=[
            pl.BlockSpec((1, S), lambda m: (0, 0)),
            pl.BlockSpec((1, S), lambda m: (0, 0)),
            pl.BlockSpec((S, C), lambda m: (0, 0)),
        ],
        out_specs=pl.BlockSpec((GBLK, C), lambda m: (m, 0)),
        out_shape=jax.ShapeDtypeStruct((N_BUF, C), f32),
    )(pos0r, pos1r, x_flat)

    w1t = jnp.transpose(W1, (2, 0, 1))
    w3t = jnp.transpose(W3, (2, 0, 1))
    w2t = jnp.transpose(W2, (1, 0, 2))

    def _expert_body(pf_ref, xs_ref, w1_hbm, w3_hbm, w2_hbm, o_ref,
                     w1b, w3b, w2b, w1ts, w3ts, sems):
        m = pl.program_id(0)
        slot = pf_ref[3, m] % 2

        def _issue(e, s):
            pltpu.make_async_copy(w1_hbm.at[:, e, :], w1b.at[s],
                                  sems.at[s]).start()
            pltpu.make_async_copy(w3_hbm.at[:, e, :], w3b.at[s],
                                  sems.at[s]).start()
            pltpu.make_async_copy(w2_hbm.at[:, e, :], w2b.at[s],
                                  sems.at[s]).start()

        def _wait(s):
            pltpu.make_async_copy(w1_hbm.at[:, 0, :], w1b.at[s],
                                  sems.at[s]).wait()
            pltpu.make_async_copy(w3_hbm.at[:, 0, :], w3b.at[s],
                                  sems.at[s]).wait()
            pltpu.make_async_copy(w2_hbm.at[:, 0, :], w2b.at[s],
                                  sems.at[s]).wait()

        @pl.when(m == 0)
        def _():
            _issue(pf_ref[4, 0], 0)

        @pl.when(pf_ref[0, m] == 1)
        def _():
            _wait(slot)

            @pl.when(pf_ref[1, m] == 1)
            def _():
                _issue(pf_ref[2, m], 1 - slot)
            w1ts[...] = w1b[slot].T
            w3ts[...] = w3b[slot].T

        xb = xs_ref[...]
        dn_n = (((1,), (0,)), ((), ()))
        h1 = lax.dot_general(xb, w1ts[...], dn_n,
                             preferred_element_type=f32)
        h3 = lax.dot_general(xb, w3ts[...], dn_n,
                             preferred_element_type=f32)
        glu = h1 * _sigmoid(h1) * h3
        o_ref[...] = lax.dot_general(glu, w2b[slot], dn_n,
                                     preferred_element_type=f32)

    grid_spec = pltpu.PrefetchScalarGridSpec(
        num_scalar_prefetch=1,
        grid=(N_BLK,),
        in_specs=[
            pl.BlockSpec((BLK_M, C), lambda m, pfr: (m, 0)),
            pl.BlockSpec(memory_space=pl.ANY),
            pl.BlockSpec(memory_space=pl.ANY),
            pl.BlockSpec(memory_space=pl.ANY),
        ],
        out_specs=pl.BlockSpec((BLK_M, C), lambda m, pfr: (m, 0)),
        scratch_shapes=[
            pltpu.VMEM((2, H, C), f32),
            pltpu.VMEM((2, H, C), f32),
            pltpu.VMEM((2, H, C), f32),
            pltpu.VMEM((C, H), f32),
            pltpu.VMEM((C, H), f32),
            pltpu.SemaphoreType.DMA((2,)),
        ],
    )
    y_sorted = pl.pallas_call(
        _expert_body,
        grid_spec=grid_spec,
        out_shape=jax.ShapeDtypeStruct((N_BUF, C), f32),
    )(pf, x_sorted, w1t, w3t, w2t)

    pos01 = jnp.concatenate([pos0, pos1])
    g = _gather_rows(y_sorted, pos01)

    ws1t = Ws1.T
    ws3t = Ws3.T

    def _shared_body(x_ref, ws1_ref, ws3_ref, ws2_ref, o_ref):
        xb = x_ref[...]
        dn_t = (((1,), (1,)), ((), ()))
        h1 = lax.dot_general(xb, ws1_ref[...], dn_t,
                             preferred_element_type=f32)
        h3 = lax.dot_general(xb, ws3_ref[...], dn_t,
                             preferred_element_type=f32)
        glu = h1 * _sigmoid(h1) * h3
        o_ref[...] = jnp.dot(glu, ws2_ref[...], preferred_element_type=f32)

    shared_out = pl.pallas_call(
        _shared_body,
        grid=(S // TOK_BLK,),
        in_specs=[
            pl.BlockSpec((TOK_BLK, C), lambda m: (m, 0)),
            pl.BlockSpec((H, C), lambda m: (0, 0)),
            pl.BlockSpec((H, C), lambda m: (0, 0)),
            pl.BlockSpec((H, C), lambda m: (0, 0)),
        ],
        out_specs=pl.BlockSpec((TOK_BLK, C), lambda m: (m, 0)),
        out_shape=jax.ShapeDtypeStruct((S, C), f32),
    )(x_flat, ws1t, ws3t, Ws2)

    def _combine_body(sh_ref, g0_ref, g1_ref, wf_ref, o_ref):
        wf = wf_ref[...]
        o_ref[...] = (sh_ref[...] + wf[:, 0:1] * g0_ref[...]
                      + wf[:, 1:2] * g1_ref[...])

    out = pl.pallas_call(
        _combine_body,
        grid=(S // TOK_BLK,),
        in_specs=[
            pl.BlockSpec((TOK_BLK, C), lambda m: (m, 0)),
            pl.BlockSpec((TOK_BLK, C), lambda m: (m, 0)),
            pl.BlockSpec((TOK_BLK, C), lambda m: (m + S // TOK_BLK, 0)),
            pl.BlockSpec((TOK_BLK, LANE), lambda m: (m, 0)),
        ],
        out_specs=pl.BlockSpec((TOK_BLK, C), lambda m: (m, 0)),
        out_shape=jax.ShapeDtypeStruct((S, C), f32),
    )(shared_out, g, g, w_pad)

    return out.reshape(Bb, Tt, C), aux_loss
